# baseline (speedup 1.0000x reference)
import jax
import jax.numpy as jnp
from jax import lax
from jax.experimental import pallas as pl
from jax.experimental.pallas import tpu as pltpu
from jax.experimental.pallas import tpu_sc as plsc

NC = 2
NS = 16
NW = NC * NS
CHUNK = 128
LANES = 16


def _sc_mesh():
    return plsc.VectorSubcoreMesh(core_axis_name="c", subcore_axis_name="s")


def _sc_gather_ts(memory, lu_blk, ids, ts):
    M2, _ = lu_blk.shape
    M, D = memory.shape
    B = ids.shape[0]
    k = B // NW // CHUNK
    bpw = k * CHUNK
    srows = M2 // NW
    spt = srows * CHUNK
    R = 2048
    G = B // R

    def body(mem_hbm, lublk_hbm, ids_hbm, ts_hbm, h_out, parts_out, lublk_out,
             idx_v, rows_v, ids_sc, ts_sc, slice_v, part_v,
             sem_r, sem_s):
        c = lax.axis_index("c")
        s = lax.axis_index("s")
        w = s * NC + c
        base = w * bpw
        lo = w * spt
        scan_copies = [
            pltpu.async_copy(lublk_hbm.at[pl.ds(w * srows, srows)], slice_v,
                             sem_s),
            pltpu.async_copy(ids_hbm, ids_sc, sem_s),
            pltpu.async_copy(ts_hbm, ts_sc, sem_s),
        ]
        for j in range(k):
            pltpu.sync_copy(ids_hbm.at[pl.ds(base + j * CHUNK, CHUNK)],
                            idx_v.at[j])
        row_copies = [
            pltpu.async_copy(mem_hbm.at[idx_v.at[j]],
                             rows_v.at[pl.ds(j * CHUNK, CHUNK)], sem_r)
            for j in range(k)
        ]
        for cp in scan_copies:
            cp.wait()

        @plsc.parallel_loop(0, B // LANES, unroll=16)
        def _(g):
            ids16 = ids_sc[pl.ds(g * LANES, LANES)]
            m = (ids16 >= lo) & (ids16 < lo + spt)
            loc = jnp.minimum(jnp.maximum(ids16 - lo, 0), spt - 1)
            locr = lax.shift_right_logical(loc, 7)
            locc = loc & 127
            cur = plsc.load_gather(slice_v, [locr, locc], mask=m)
            part_v[lax.shift_right_logical(g, 7), 0,
                   pl.ds((g & 127) * LANES, LANES)] = jnp.where(m, cur, 0.0)
            newv = jnp.maximum(ts_sc[pl.ds(g * LANES, LANES)], cur)
            plsc.store_scatter(slice_v, [locr, locc], newv, mask=m)

        pltpu.sync_copy(slice_v, lublk_out.at[pl.ds(w * srows, srows)])
        pltpu.sync_copy(part_v, parts_out.at[w])
        for cp in row_copies:
            cp.wait()
        pltpu.sync_copy(rows_v, h_out.at[pl.ds(base, bpw)])

    f = pl.kernel(
        body,
        out_type=(jax.ShapeDtypeStruct((B, D), jnp.float32),
                  jax.ShapeDtypeStruct((NW, G, 1, R), jnp.float32),
                  jax.ShapeDtypeStruct((M2, CHUNK), jnp.float32)),
        mesh=_sc_mesh(),
        scratch_types=[
            pltpu.VMEM((k, CHUNK), jnp.int32),
            pltpu.VMEM((bpw, D), jnp.float32),
            pltpu.VMEM((B,), jnp.int32),
            pltpu.VMEM((B,), jnp.float32),
            pltpu.VMEM((srows, CHUNK), jnp.float32),
            pltpu.VMEM((G, 1, R), jnp.float32),
            pltpu.SemaphoreType.DMA,
            pltpu.SemaphoreType.DMA,
        ],
        compiler_params=pltpu.CompilerParams(needs_layout_passes=False,
                                             use_tc_tiling_on_sc=True),
        name="sc_gather_ts",
    )
    return f(memory, lu_blk, ids, ts)


def _sc_scatter(mem_ref, ids, upd_rows):
    B, D = upd_rows.shape
    k = B // NW // CHUNK
    bpw = k * CHUNK

    def body(mem_hbm, ids_hbm, rows_hbm, idx_v, rows_v, sem_m):
        c = lax.axis_index("c")
        s = lax.axis_index("s")
        w = s * NC + c
        base = w * bpw
        stage = [
            pltpu.async_copy(rows_hbm.at[pl.ds(base, bpw)], rows_v, sem_m)
        ] + [
            pltpu.async_copy(ids_hbm.at[pl.ds(base + j * CHUNK, CHUNK)],
                             idx_v.at[j], sem_m)
            for j in range(k)
        ]
        for cp in stage:
            cp.wait()
        row_copies = [
            pltpu.async_copy(rows_v.at[pl.ds(j * CHUNK, CHUNK)],
                             mem_hbm.at[idx_v.at[j]], sem_m)
            for j in range(k)
        ]
        for cp in row_copies:
            cp.wait()

    f = pl.kernel(
        body,
        out_type=(),
        mesh=_sc_mesh(),
        scratch_types=[
            pltpu.VMEM((k, CHUNK), jnp.int32),
            pltpu.VMEM((bpw, D), jnp.float32),
            pltpu.SemaphoreType.DMA,
        ],
        compiler_params=pltpu.CompilerParams(needs_layout_passes=False,
                                             use_tc_tiling_on_sc=True),
        name="sc_scatter_rows",
    )
    f(mem_ref, ids, upd_rows)


def _tc_copy_gru(memory, messages, h, parts, ts3, W_ih, W_hh, b_ih, b_hh):
    M, D = memory.shape
    B, DM = messages.shape
    R = 2048
    G = B // R
    CB = 16384
    NB = M // CB
    bih2 = b_ih.reshape(1, 3 * D)
    bhh2 = b_hh.reshape(1, 3 * D)

    NWL = parts.shape[0]

    def body(mem_ref, x_ref, h_ref, parts_ref, ts_ref, wih_ref, whh_ref,
             bih_ref, bhh_ref, out_mem_ref, upd_ref):
        i = pl.program_id(0)
        out_mem_ref[...] = mem_ref[...]

        @pl.when(i < G)
        def _():
            x = x_ref[...]
            hh = h_ref[...]
            dn = (((1,), (1,)), ((), ()))
            gi = lax.dot_general(
                x, wih_ref[...], dimension_numbers=dn,
                preferred_element_type=jnp.float32) + bih_ref[...]
            gh = lax.dot_general(
                hh, whh_ref[...], dimension_numbers=dn,
                preferred_element_type=jnp.float32) + bhh_ref[...]
            r = jax.nn.sigmoid(gi[:, :D] + gh[:, :D])
            z = jax.nn.sigmoid(gi[:, D:2 * D] + gh[:, D:2 * D])
            n = jnp.tanh(gi[:, 2 * D:] + r * gh[:, 2 * D:])
            h_new = (1.0 - z) * n + z * hh
            pv = parts_ref[:, 0, 0, :]
            lu_row = jnp.sum(pv, axis=0, keepdims=True)
            d_row = ts_ref[0] - lu_row
            d_col = jnp.transpose(d_row, (1, 0))
            valid = d_col >= 0.0
            upd_ref[...] = jnp.where(valid, h_new, hh)

    return pl.pallas_call(
        body,
        grid=(NB,),
        in_specs=[
            pl.BlockSpec((CB, D), lambda i: (i, 0)),
            pl.BlockSpec((R, DM), lambda i: (jnp.minimum(i, G - 1), 0)),
            pl.BlockSpec((R, D), lambda i: (jnp.minimum(i, G - 1), 0)),
            pl.BlockSpec((parts.shape[0], 1, 1, R),
                         lambda i: (0, jnp.minimum(i, G - 1), 0, 0)),
            pl.BlockSpec((1, 1, R), lambda i: (jnp.minimum(i, G - 1), 0, 0)),
            pl.BlockSpec(W_ih.shape, lambda i: (0, 0)),
            pl.BlockSpec(W_hh.shape, lambda i: (0, 0)),
            pl.BlockSpec((1, 3 * D), lambda i: (0, 0)),
            pl.BlockSpec((1, 3 * D), lambda i: (0, 0)),
        ],
        out_specs=[
            pl.BlockSpec((CB, D), lambda i: (i, 0)),
            pl.BlockSpec((R, D), lambda i: (jnp.minimum(i, G - 1), 0)),
        ],
        out_shape=[
            jax.ShapeDtypeStruct((M, D), jnp.float32),
            jax.ShapeDtypeStruct((B, D), jnp.float32),
        ],
        name="tc_copy_gru",
    )(memory, messages, h, parts, ts3, W_ih, W_hh, bih2, bhh2)


def kernel(memory, last_update, unique_node_ids, unique_messages, timestamps,
           W_ih, W_hh, b_ih, b_hh):
    M = memory.shape[0]
    B = unique_messages.shape[0]
    ids = unique_node_ids.astype(jnp.int32)
    lu_blk = last_update.reshape(M // 128, 128)

    h, lu_parts, lu_out_blk = _sc_gather_ts(memory, lu_blk, ids, timestamps)
    R = 2048
    G = B // R
    ts3 = timestamps.reshape(G, 1, R)
    out_mem0, upd_rows = _tc_copy_gru(memory, unique_messages, h, lu_parts,
                                      ts3, W_ih, W_hh, b_ih, b_hh)

    mem_ref = jax.new_ref(out_mem0)
    _sc_scatter(mem_ref, ids, upd_rows)
    return mem_ref[...], lu_out_blk.reshape(M)


# speedup vs baseline: 1.0602x; 1.0602x over previous
import jax
import jax.numpy as jnp
from jax import lax
from jax.experimental import pallas as pl
from jax.experimental.pallas import tpu as pltpu
from jax.experimental.pallas import tpu_sc as plsc

NC = 2
NS = 16
NW = NC * NS
CHUNK = 128
LANES = 16


def _sc_mesh():
    return plsc.VectorSubcoreMesh(core_axis_name="c", subcore_axis_name="s")


def _sc_gather_ts(memory, lu_blk, ids, ts):
    M2, _ = lu_blk.shape
    M, D = memory.shape
    B = ids.shape[0]
    k = B // NW // CHUNK
    bpw = k * CHUNK
    srows = M2 // NW
    spt = srows * CHUNK
    R = 2048
    G = B // R

    def body(mem_hbm, lublk_hbm, ids_hbm, ts_hbm, h_out, parts_out, lublk_out,
             idx_v, rows_v, ids_sc, ts_sc, slice_v, part_v,
             sem_r, sem_s):
        c = lax.axis_index("c")
        s = lax.axis_index("s")
        w = s * NC + c
        base = w * bpw
        lo = w * spt
        scan_copies = [
            pltpu.async_copy(lublk_hbm.at[pl.ds(w * srows, srows)], slice_v,
                             sem_s),
            pltpu.async_copy(ids_hbm, ids_sc, sem_s),
            pltpu.async_copy(ts_hbm, ts_sc, sem_s),
        ]
        for j in range(k):
            pltpu.sync_copy(ids_hbm.at[pl.ds(base + j * CHUNK, CHUNK)],
                            idx_v.at[j])
        row_copies = [
            pltpu.async_copy(mem_hbm.at[idx_v.at[j]],
                             rows_v.at[pl.ds(j * CHUNK, CHUNK)], sem_r)
            for j in range(k)
        ]
        for cp in scan_copies:
            cp.wait()

        @plsc.parallel_loop(0, B // LANES, unroll=8)
        def _(g):
            ids16 = ids_sc[pl.ds(g * LANES, LANES)]
            m = (ids16 >= lo) & (ids16 < lo + spt)
            loc = jnp.minimum(jnp.maximum(ids16 - lo, 0), spt - 1)
            locr = lax.shift_right_logical(loc, 7)
            locc = loc & 127
            cur = plsc.load_gather(slice_v, [locr, locc], mask=m)
            part_v[lax.shift_right_logical(g, 7), 0,
                   pl.ds((g & 127) * LANES, LANES)] = jnp.where(m, cur, 0.0)
            newv = jnp.maximum(ts_sc[pl.ds(g * LANES, LANES)], cur)
            plsc.store_scatter(slice_v, [locr, locc], newv, mask=m)

        pltpu.sync_copy(slice_v, lublk_out.at[pl.ds(w * srows, srows)])
        pltpu.sync_copy(part_v, parts_out.at[w])
        for cp in row_copies:
            cp.wait()
        pltpu.sync_copy(rows_v, h_out.at[pl.ds(base, bpw)])

    f = pl.kernel(
        body,
        out_type=(jax.ShapeDtypeStruct((B, D), jnp.float32),
                  jax.ShapeDtypeStruct((NW, G, 1, R), jnp.float32),
                  jax.ShapeDtypeStruct((M2, CHUNK), jnp.float32)),
        mesh=_sc_mesh(),
        scratch_types=[
            pltpu.VMEM((k, CHUNK), jnp.int32),
            pltpu.VMEM((bpw, D), jnp.float32),
            pltpu.VMEM((B,), jnp.int32),
            pltpu.VMEM((B,), jnp.float32),
            pltpu.VMEM((srows, CHUNK), jnp.float32),
            pltpu.VMEM((G, 1, R), jnp.float32),
            pltpu.SemaphoreType.DMA,
            pltpu.SemaphoreType.DMA,
        ],
        compiler_params=pltpu.CompilerParams(needs_layout_passes=False,
                                             use_tc_tiling_on_sc=True),
        name="sc_gather_ts",
    )
    return f(memory, lu_blk, ids, ts)


def _sc_scatter(mem_ref, ids, upd_rows):
    B, D = upd_rows.shape
    k = B // NW // CHUNK
    bpw = k * CHUNK

    def body(mem_hbm, ids_hbm, rows_hbm, idx_v, rows_v, sem_m):
        c = lax.axis_index("c")
        s = lax.axis_index("s")
        w = s * NC + c
        base = w * bpw
        stage = [
            pltpu.async_copy(rows_hbm.at[pl.ds(base, bpw)], rows_v, sem_m)
        ] + [
            pltpu.async_copy(ids_hbm.at[pl.ds(base + j * CHUNK, CHUNK)],
                             idx_v.at[j], sem_m)
            for j in range(k)
        ]
        for cp in stage:
            cp.wait()
        row_copies = [
            pltpu.async_copy(rows_v.at[pl.ds(j * CHUNK, CHUNK)],
                             mem_hbm.at[idx_v.at[j]], sem_m)
            for j in range(k)
        ]
        for cp in row_copies:
            cp.wait()

    f = pl.kernel(
        body,
        out_type=(),
        mesh=_sc_mesh(),
        scratch_types=[
            pltpu.VMEM((k, CHUNK), jnp.int32),
            pltpu.VMEM((bpw, D), jnp.float32),
            pltpu.SemaphoreType.DMA,
        ],
        compiler_params=pltpu.CompilerParams(needs_layout_passes=False,
                                             use_tc_tiling_on_sc=True),
        name="sc_scatter_rows",
    )
    f(mem_ref, ids, upd_rows)


def _tc_copy_gru(memory, messages, h, parts, ts3, W_ih, W_hh, b_ih, b_hh):
    M, D = memory.shape
    B, DM = messages.shape
    R = 2048
    G = B // R
    CB = 16384
    NB = M // CB
    bih2 = b_ih.reshape(1, 3 * D)
    bhh2 = b_hh.reshape(1, 3 * D)

    NWL = parts.shape[0]

    def body(mem_ref, x_ref, h_ref, parts_ref, ts_ref, wih_ref, whh_ref,
             bih_ref, bhh_ref, out_mem_ref, upd_ref):
        i = pl.program_id(0)
        out_mem_ref[...] = mem_ref[...]

        @pl.when(i < G)
        def _():
            x = x_ref[...]
            hh = h_ref[...]
            dn = (((1,), (1,)), ((), ()))
            gi = lax.dot_general(
                x, wih_ref[...], dimension_numbers=dn,
                preferred_element_type=jnp.float32) + bih_ref[...]
            gh = lax.dot_general(
                hh, whh_ref[...], dimension_numbers=dn,
                preferred_element_type=jnp.float32) + bhh_ref[...]
            r = jax.nn.sigmoid(gi[:, :D] + gh[:, :D])
            z = jax.nn.sigmoid(gi[:, D:2 * D] + gh[:, D:2 * D])
            n = jnp.tanh(gi[:, 2 * D:] + r * gh[:, 2 * D:])
            h_new = (1.0 - z) * n + z * hh
            pv = parts_ref[:, 0, 0, :]
            lu_row = jnp.sum(pv, axis=0, keepdims=True)
            d_row = ts_ref[0] - lu_row
            d_col = jnp.transpose(d_row, (1, 0))
            valid = d_col >= 0.0
            upd_ref[...] = jnp.where(valid, h_new, hh)

    return pl.pallas_call(
        body,
        grid=(NB,),
        in_specs=[
            pl.BlockSpec((CB, D), lambda i: (i, 0)),
            pl.BlockSpec((R, DM), lambda i: (jnp.minimum(i, G - 1), 0)),
            pl.BlockSpec((R, D), lambda i: (jnp.minimum(i, G - 1), 0)),
            pl.BlockSpec((parts.shape[0], 1, 1, R),
                         lambda i: (0, jnp.minimum(i, G - 1), 0, 0)),
            pl.BlockSpec((1, 1, R), lambda i: (jnp.minimum(i, G - 1), 0, 0)),
            pl.BlockSpec(W_ih.shape, lambda i: (0, 0)),
            pl.BlockSpec(W_hh.shape, lambda i: (0, 0)),
            pl.BlockSpec((1, 3 * D), lambda i: (0, 0)),
            pl.BlockSpec((1, 3 * D), lambda i: (0, 0)),
        ],
        out_specs=[
            pl.BlockSpec((CB, D), lambda i: (i, 0)),
            pl.BlockSpec((R, D), lambda i: (jnp.minimum(i, G - 1), 0)),
        ],
        out_shape=[
            jax.ShapeDtypeStruct((M, D), jnp.float32),
            jax.ShapeDtypeStruct((B, D), jnp.float32),
        ],
        name="tc_copy_gru",
    )(memory, messages, h, parts, ts3, W_ih, W_hh, bih2, bhh2)


def kernel(memory, last_update, unique_node_ids, unique_messages, timestamps,
           W_ih, W_hh, b_ih, b_hh):
    M = memory.shape[0]
    B = unique_messages.shape[0]
    ids = unique_node_ids.astype(jnp.int32)
    lu_blk = last_update.reshape(M // 128, 128)

    h, lu_parts, lu_out_blk = _sc_gather_ts(memory, lu_blk, ids, timestamps)
    R = 2048
    G = B // R
    ts3 = timestamps.reshape(G, 1, R)
    out_mem0, upd_rows = _tc_copy_gru(memory, unique_messages, h, lu_parts,
                                      ts3, W_ih, W_hh, b_ih, b_hh)

    mem_ref = jax.new_ref(out_mem0)
    _sc_scatter(mem_ref, ids, upd_rows)
    return mem_ref[...], lu_out_blk.reshape(M)
